# baseline (speedup 1.0000x reference)
import functools

import jax
import jax.numpy as jnp
from jax import lax
from jax.experimental import pallas as pl
from jax.experimental.pallas import tpu as pltpu
from jax.experimental.pallas import tpu_sc as plsc


_CH = 8


def _sc_add_kernel(batch, seq_len, dim, x_hbm, w_hbm, out_hbm, xbuf, wbuf, sem):
    nc = 2
    ns = 16
    wid = lax.axis_index("s") * nc + lax.axis_index("c")
    rows_per_worker = seq_len // (nc * ns)
    s_base = wid * rows_per_worker
    n_chunks = rows_per_worker // _CH
    vecs_per_row = dim // 16
    for b in range(batch):
        def body(j, carry):
            s0 = s_base + j * _CH
            pltpu.sync_copy(x_hbm.at[b, pl.ds(s0, _CH)], xbuf)
            pltpu.sync_copy(w_hbm.at[pl.ds(s0, _CH)], wbuf)

            def add_row(r, c2):
                def add_vec(v, c3):
                    off = v * 16
                    xbuf[r, pl.ds(off, 16)] = (
                        xbuf[r, pl.ds(off, 16)] + wbuf[r, pl.ds(off, 16)]
                    )
                    return c3
                lax.fori_loop(0, vecs_per_row, add_vec, 0)
                return c2
            lax.fori_loop(0, _CH, add_row, 0)

            pltpu.sync_copy(xbuf, out_hbm.at[b, pl.ds(s0, _CH)])
            return carry
        lax.fori_loop(0, n_chunks, body, 0)


def kernel(inputs, weight):
    batch, seq_len, dim = inputs.shape
    mesh = plsc.VectorSubcoreMesh(core_axis_name="c", subcore_axis_name="s")
    k = pl.kernel(
        functools.partial(_sc_add_kernel, batch, seq_len, dim),
        out_type=jax.ShapeDtypeStruct((batch, seq_len, dim), inputs.dtype),
        mesh=mesh,
        scratch_types=[
            pltpu.VMEM((_CH, dim), jnp.float32),
            pltpu.VMEM((_CH, dim), jnp.float32),
            pltpu.SemaphoreType.DMA,
        ],
    )
    return k(inputs, weight)


# speedup vs baseline: 2.4960x; 2.4960x over previous
import functools

import jax
import jax.numpy as jnp
from jax import lax
from jax.experimental import pallas as pl
from jax.experimental.pallas import tpu as pltpu
from jax.experimental.pallas import tpu_sc as plsc


_CH = 8


def _sc_add_kernel(batch, seq_len, dim, x_hbm, w_hbm, out_hbm, xbuf, wbuf, sem):
    nc = 2
    ns = 16
    wid = lax.axis_index("s") * nc + lax.axis_index("c")
    rows_per_worker = seq_len // (nc * ns)
    s_base = wid * rows_per_worker
    n_chunks = rows_per_worker // _CH
    vecs_per_row = dim // 16
    for b in range(batch):
        def body(j, carry):
            s0 = s_base + j * _CH
            pltpu.sync_copy(x_hbm.at[b, pl.ds(s0, _CH)], xbuf)
            pltpu.sync_copy(w_hbm.at[pl.ds(s0, _CH)], wbuf)

            def add_row(r, c2):
                @functools.partial(
                    plsc.parallel_loop, 0, vecs_per_row, unroll=8
                )
                def add_vec(v):
                    off = v * 16
                    plsc.addupdate(
                        xbuf.at[r, pl.ds(off, 16)], wbuf[r, pl.ds(off, 16)]
                    )
                return c2
            lax.fori_loop(0, _CH, add_row, 0)

            pltpu.sync_copy(xbuf, out_hbm.at[b, pl.ds(s0, _CH)])
            return carry
        lax.fori_loop(0, n_chunks, body, 0)


def kernel(inputs, weight):
    batch, seq_len, dim = inputs.shape
    mesh = plsc.VectorSubcoreMesh(core_axis_name="c", subcore_axis_name="s")
    k = pl.kernel(
        functools.partial(_sc_add_kernel, batch, seq_len, dim),
        out_type=jax.ShapeDtypeStruct((batch, seq_len, dim), inputs.dtype),
        mesh=mesh,
        scratch_types=[
            pltpu.VMEM((_CH, dim), jnp.float32),
            pltpu.VMEM((_CH, dim), jnp.float32),
            pltpu.SemaphoreType.DMA,
        ],
    )
    return k(inputs, weight)
